# baseline (speedup 1.0000x reference)
import functools
import jax
import jax.numpy as jnp
from jax import lax
from jax.experimental import pallas as pl
from jax.experimental.pallas import tpu as pltpu
from jax.experimental.pallas import tpu_sc as plsc


_BIG = 1e6


def _chamfer_all_kernel(gt_ref, p_ref,
                        d1a_ref, d1b_ref, d1c_ref, d2a_ref, d2b_ref, d2c_ref,
                        key_ref, *, nsteps, NT, segs, Vcat):
    b = pl.program_id(0)
    ni = pl.program_id(1)
    g = gt_ref[0]
    p = p_ref[0]
    aa = jnp.sum(g * g, axis=1, keepdims=True)
    ghat = jnp.concatenate([-2.0 * g, aa, jnp.ones_like(aa)], axis=1)
    D = jnp.maximum(
        jnp.dot(ghat, p, preferred_element_type=jnp.float32), 0.0)
    d1refs = (d1a_ref, d1b_ref, d1c_ref)
    d2refs = (d2a_ref, d2b_ref, d2c_ref)

    @pl.when((b == 0) & (ni == 0))
    def _():
        for r in d1refs + d2refs:
            r[:, :] = jnp.zeros((1, 1), jnp.float32)

    for l, (o, w, V) in enumerate(segs):
        d1t = jnp.min(D[:, o:o + w], axis=1, keepdims=True)
        d1refs[l][:, :] += jnp.sum(jnp.sqrt(d1t + 1e-12), keepdims=True)

    rowi = lax.broadcasted_iota(jnp.int32, (NT, 1), 0) + ni * NT
    key = (lax.bitcast_convert_type(D, jnp.int32) & jnp.int32(-2048)) | rowi
    tkey = jnp.min(key, axis=0, keepdims=True)

    @pl.when(ni == 0)
    def _():
        key_ref[0] = tkey

    @pl.when(ni > 0)
    def _():
        key_ref[0] = jnp.minimum(key_ref[0], tkey)

    @pl.when(ni == nsteps - 1)
    def _():
        viota = lax.broadcasted_iota(jnp.int32, (1, Vcat), 1)
        kf = lax.bitcast_convert_type(key_ref[0] & jnp.int32(-2048),
                                      jnp.float32)
        d2sq = jnp.sqrt(kf + 1e-12)
        for l, (o, w, V) in enumerate(segs):
            valid = (viota[:, o:o + w] - o) < V
            seg = jnp.where(valid, d2sq[:, o:o + w], 0.0)
            d2refs[l][:, :] += jnp.sum(seg, keepdims=True)


def _chamfer_all(gt, preds):
    B, N, _ = gt.shape
    NT = 512
    nsteps = N // NT
    segs = []
    cats = []
    off = 0
    for P in preds:
        V = P.shape[1]
        Vp = max(256, ((V + 127) // 128) * 128)
        Pp = jnp.pad(P, ((0, 0), (0, Vp - V), (0, 0)), constant_values=_BIG)
        cats.append(jnp.transpose(Pp, (0, 2, 1)))
        segs.append((off, Vp, V))
        off += Vp
    Vcat = off
    Pt = jnp.concatenate(cats, axis=2)
    bb = jnp.sum(Pt * Pt, axis=1, keepdims=True)
    Phat = jnp.concatenate([Pt, jnp.ones_like(bb), bb], axis=1)

    kern = functools.partial(_chamfer_all_kernel, nsteps=nsteps, NT=NT,
                             segs=tuple(segs), Vcat=Vcat)
    scal = pl.BlockSpec((1, 1), lambda b, n: (0, 0))
    outs = pl.pallas_call(
        kern,
        grid=(B, nsteps),
        in_specs=[
            pl.BlockSpec((1, NT, 3), lambda b, n: (b, n, 0)),
            pl.BlockSpec((1, 5, Vcat), lambda b, n: (b, 0, 0)),
        ],
        out_specs=[scal] * 6 + [
            pl.BlockSpec((1, 1, Vcat), lambda b, n: (b, 0, 0)),
        ],
        out_shape=[jax.ShapeDtypeStruct((1, 1), jnp.float32)] * 6 + [
            jax.ShapeDtypeStruct((B, 1, Vcat), jnp.int32),
        ],
    )(gt, Phat)
    d1_sums = [outs[l][0, 0] for l in range(3)]
    d2_sums = [outs[3 + l][0, 0] for l in range(3)]
    key_cat = outs[6][:, 0, :]
    return d1_sums, d2_sums, key_cat


_LVL = [
    dict(V=156, E=462, NCH=2, WBASE=24, EC=240, VC=80, Vr=160, Er=480, O=0),
    dict(V=618, E=1848, NCH=2, WBASE=16, EC=928, VC=320, Vr=640, Er=1856,
         O=256),
    dict(V=2466, E=7392, NCH=4, WBASE=0, EC=1856, VC=624, Vr=2496, Er=7424,
         O=896),
]
_VR_MAX = 2496
_EC_MAX = 1856
_VC_MAX = 624
_NB = 4
_NPTS = 2048
_VCAT = 3456

_off = 0
_PO = []
for _c in _LVL:
    _PO.append(_off)
    _off += 3 * _NB * _c["Vr"]
_BO = []
for _c in _LVL:
    _BO.append(_off)
    _off += 3 * _NB * _c["Vr"]
_NO = _off
_F_TOT = _NO + 3 * _NB * _NPTS
_off = 0
_LNO = []
for _c in _LVL:
    _LNO.append(_off)
    _off += 8 * _c["Vr"]
_LCO = []
for _c in _LVL:
    _LCO.append(_off)
    _off += _c["Vr"]
_ETO = []
for _c in _LVL:
    _ETO.append(_off)
    _off += 2 * _c["Er"]
_I_TOT = _off


def _rsqrt16(x):
    xi = plsc.bitcast(x, jnp.int32)
    yi = jnp.int32(0x5F3759DF) - (xi >> 1)
    y = plsc.bitcast(yi, jnp.float32)
    for _ in range(4):
        y = y * (1.5 - 0.5 * x * y * y)
    return y


def _reg_branch(cfg, lvl, wid, fb, ib, kf,
                out, pvs, bvs, dvs, nvs, i2v, e0v, e1v, lnv, lcv, ov, sem):
    V, E, NCH, WBASE = cfg["V"], cfg["E"], cfg["NCH"], cfg["WBASE"]
    EC, VC, Vr, Er, O = cfg["EC"], cfg["VC"], cfg["Vr"], cfg["Er"], cfg["O"]
    PO, BO = _PO[lvl], _BO[lvl]
    LN, LC, ET = _LNO[lvl], _LCO[lvl], _ETO[lvl]
    B, N = _NB, _NPTS
    t = wid - WBASE
    b = t // NCH
    chunk = t % NCH

    cps = []
    for c in range(3):
        cps.append(pltpu.async_copy(fb.at[pl.ds(PO + (c * B + b) * Vr, Vr)],
                                    pvs[c].at[pl.ds(0, Vr)], sem))
        cps.append(pltpu.async_copy(fb.at[pl.ds(BO + (c * B + b) * Vr, Vr)],
                                    bvs[c].at[pl.ds(0, Vr)], sem))
        cps.append(pltpu.async_copy(fb.at[pl.ds(_NO + (c * B + b) * N, N)],
                                    nvs[c], sem))
    cps.append(pltpu.async_copy(kf.at[pl.ds(b * _VCAT + O, Vr)],
                                i2v.at[pl.ds(0, Vr)], sem))
    cps.append(pltpu.async_copy(ib.at[pl.ds(ET + chunk * EC, EC)],
                                e0v.at[pl.ds(0, EC)], sem))
    cps.append(pltpu.async_copy(ib.at[pl.ds(ET + Er + chunk * EC, EC)],
                                e1v.at[pl.ds(0, EC)], sem))
    for k in range(8):
        cps.append(pltpu.async_copy(
            ib.at[pl.ds(LN + k * Vr + chunk * VC, VC)],
            lnv.at[pl.ds(k * _VC_MAX, VC)], sem))
    cps.append(pltpu.async_copy(ib.at[pl.ds(LC + chunk * VC, VC)],
                                lcv.at[pl.ds(0, VC)], sem))
    for cp in cps:
        cp.wait()

    iota16 = lax.broadcasted_iota(jnp.int32, (16,), 0)
    z16 = jnp.zeros((16,), jnp.float32)

    def dv_body(i, carry):
        s = pl.ds(i * 16, 16)
        for c in range(3):
            dvs[c][s] = bvs[c][s] - pvs[c][s]
        return carry

    lax.fori_loop(0, Vr // 16, dv_body, 0)

    def e_body(i, carry):
        ae, an = carry
        s = pl.ds(i * 16, 16)
        i0 = e0v[s]
        i1 = e1v[s]
        m = (chunk * EC + i * 16 + iota16) < E
        dx = plsc.load_gather(pvs[0], [i0]) - plsc.load_gather(pvs[0], [i1])
        dy = plsc.load_gather(pvs[1], [i0]) - plsc.load_gather(pvs[1], [i1])
        dz = plsc.load_gather(pvs[2], [i0]) - plsc.load_gather(pvs[2], [i1])
        dd = dx * dx + dy * dy + dz * dz
        j = plsc.load_gather(i2v, [i0]) & 2047
        gx = plsc.load_gather(nvs[0], [j])
        gy = plsc.load_gather(nvs[1], [j])
        gz = plsc.load_gather(nvs[2], [j])
        dot = dx * gx + dy * gy + dz * gz
        nn2 = gx * gx + gy * gy + gz * gz
        cos = jnp.abs(dot) * _rsqrt16(jnp.maximum(dd * nn2, 1e-24))
        ae = ae + jnp.where(m, dd, 0.0)
        an = an + jnp.where(m, cos, 0.0)
        return ae, an

    ae, an = lax.fori_loop(0, EC // 16, e_body, (z16, z16))

    def l_body(i, carry):
        al, am = carry
        gbase = chunk * VC + i * 16
        sg = pl.ds(gbase, 16)
        m = (gbase + iota16) < V
        sx = dvs[0][sg]
        sy = dvs[1][sg]
        sz = dvs[2][sg]
        cntf = lcv[pl.ds(i * 16, 16)].astype(jnp.float32)
        accx = z16
        accy = z16
        accz = z16
        for k in range(8):
            nk = lnv[pl.ds(k * _VC_MAX + i * 16, 16)]
            mk = nk >= 0
            sk = jnp.where(mk, nk, 0)
            accx = accx + jnp.where(mk, plsc.load_gather(dvs[0], [sk]), 0.0)
            accy = accy + jnp.where(mk, plsc.load_gather(dvs[1], [sk]), 0.0)
            accz = accz + jnp.where(mk, plsc.load_gather(dvs[2], [sk]), 0.0)
        lx = sx - accx / cntf
        ly = sy - accy / cntf
        lz = sz - accz / cntf
        al = al + jnp.where(m, lx * lx + ly * ly + lz * lz, 0.0)
        am = am + jnp.where(m, sx * sx + sy * sy + sz * sz, 0.0)
        return al, am

    al, am = lax.fori_loop(0, VC // 16, l_body, (z16, z16))

    vec = (jnp.where(iota16 == lvl, jnp.sum(ae), 0.0)
           + jnp.where(iota16 == 3 + lvl, jnp.sum(an), 0.0)
           + jnp.where(iota16 == 6 + lvl, jnp.sum(al), 0.0)
           + jnp.where(iota16 == 9 + lvl, jnp.sum(am), 0.0))
    ov[...] = vec
    pltpu.sync_copy(ov, out.at[pl.ds(wid * 16, 16)])


def _reg_body(fb, ib, kf, out,
              pvx, pvy, pvz, bvx, bvy, bvz, dvx, dvy, dvz,
              nvx, nvy, nvz, i2v, e0v, e1v, lnv, lcv, ov, sem):
    wid = lax.axis_index("c") * 16 + lax.axis_index("s")
    pvs = (pvx, pvy, pvz)
    bvs = (bvx, bvy, bvz)
    dvs = (dvx, dvy, dvz)
    nvs = (nvx, nvy, nvz)
    for lvl in range(3):
        cfg = _LVL[lvl]
        lo = cfg["WBASE"]
        hi = lo + 4 * cfg["NCH"]

        @pl.when((wid >= lo) & (wid < hi))
        def _(lvl=lvl, cfg=cfg):
            _reg_branch(cfg, lvl, wid, fb, ib, kf, out,
                        pvs, bvs, dvs, nvs, i2v, e0v, e1v, lnv, lcv, ov, sem)


def _reg_sc(preds, befores, gt_normals, key_cat, laps, edges_l):
    fparts = []
    iparts = []
    for lvl in range(3):
        cfg = _LVL[lvl]
        V, Vr = cfg["V"], cfg["Vr"]
        P = preds[lvl]
        p3 = jnp.transpose(jnp.pad(P, ((0, 0), (0, Vr - V), (0, 0))), (2, 0, 1))
        fparts.append(p3.reshape(-1))
    for lvl in range(3):
        cfg = _LVL[lvl]
        V, Vr = cfg["V"], cfg["Vr"]
        bf = befores[lvl]
        b3 = jnp.transpose(jnp.pad(bf, ((0, 0), (0, Vr - V), (0, 0))), (2, 0, 1))
        fparts.append(b3.reshape(-1))
    fparts.append(jnp.transpose(gt_normals, (2, 0, 1)).reshape(-1))
    for lvl in range(3):
        cfg = _LVL[lvl]
        V, Vr = cfg["V"], cfg["Vr"]
        iparts.append(jnp.pad(laps[lvl][:, :8].T, ((0, 0), (0, Vr - V))).reshape(-1))
    for lvl in range(3):
        cfg = _LVL[lvl]
        V, Vr = cfg["V"], cfg["Vr"]
        iparts.append(jnp.pad(laps[lvl][:, -1], (0, Vr - V), constant_values=1))
    for lvl in range(3):
        cfg = _LVL[lvl]
        E, Er = cfg["E"], cfg["Er"]
        iparts.append(jnp.pad(edges_l[lvl].T, ((0, 0), (0, Er - E))).reshape(-1))
    args = [jnp.concatenate(fparts), jnp.concatenate(iparts),
            key_cat.reshape(-1)]

    mesh = plsc.VectorSubcoreMesh(core_axis_name="c", subcore_axis_name="s",
                                  num_cores=2, num_subcores=16)
    out = pl.kernel(
        _reg_body,
        out_type=jax.ShapeDtypeStruct((512,), jnp.float32),
        mesh=mesh,
        compiler_params=pltpu.CompilerParams(needs_layout_passes=False),
        scratch_types=(
            [pltpu.VMEM((_VR_MAX,), jnp.float32)] * 9
            + [pltpu.VMEM((_NPTS,), jnp.float32)] * 3
            + [pltpu.VMEM((_VR_MAX,), jnp.int32),
               pltpu.VMEM((_EC_MAX,), jnp.int32),
               pltpu.VMEM((_EC_MAX,), jnp.int32),
               pltpu.VMEM((8 * _VC_MAX,), jnp.int32),
               pltpu.VMEM((_VC_MAX,), jnp.int32),
               pltpu.VMEM((16,), jnp.float32),
               pltpu.SemaphoreType.DMA]
        ),
    )(*args)
    sums = out.reshape(32, 16).sum(axis=0)
    return sums


def kernel(pred_0, pred_1, pred_2, before_0, before_1, before_2,
           gt_points, gt_normals, gt_images,
           lap_idx_0, lap_idx_1, lap_idx_2,
           edges_0, edges_1, edges_2):
    w_chamfer_opp = 0.55
    w_laplace, w_move, w_edge, w_normal = 0.5, 0.1, 0.1, 0.00016
    lap_const = [0.2, 1.0, 1.0]
    preds = [pred_0, pred_1, pred_2]
    befores = [before_0, before_1, before_2]
    laps = [lap_idx_0, lap_idx_1, lap_idx_2]
    edges_l = [edges_0, edges_1, edges_2]
    B, N, _ = gt_points.shape

    chamfer_loss = jnp.float32(0.0)
    d1_sums, d2_sums, key_cat = _chamfer_all(gt_points, preds)
    for lvl in range(3):
        V = preds[lvl].shape[1]
        chamfer_loss = (chamfer_loss + d1_sums[lvl] / (B * N)
                        + w_chamfer_opp * d2_sums[lvl] / (B * V))

    sums = _reg_sc(preds, befores, gt_normals, key_cat, laps, edges_l)

    edge_loss = jnp.float32(0.0)
    normal_loss = jnp.float32(0.0)
    lap_loss = jnp.float32(0.0)
    move_loss = jnp.float32(0.0)
    for lvl in range(3):
        V = preds[lvl].shape[1]
        E = edges_l[lvl].shape[0]
        edge_loss = edge_loss + sums[lvl] / (B * E)
        normal_loss = normal_loss + sums[3 + lvl] / (B * E)
        lap_loss = lap_loss + lap_const[lvl] * sums[6 + lvl] / (B * V)
        if lvl > 0:
            move_loss = move_loss + lap_const[lvl] * sums[9 + lvl] / (B * V)

    loss = (chamfer_loss + lap_loss * w_laplace + move_loss * w_move
            + edge_loss * w_edge + normal_loss * w_normal)
    return (loss, chamfer_loss, edge_loss, lap_loss, move_loss, normal_loss)


# speedup vs baseline: 1.0108x; 1.0108x over previous
import functools
import jax
import jax.numpy as jnp
from jax import lax
from jax.experimental import pallas as pl
from jax.experimental.pallas import tpu as pltpu
from jax.experimental.pallas import tpu_sc as plsc


_BIG = 1e6


def _chamfer_all_kernel(gt_ref, p_ref,
                        d1a_ref, d1b_ref, d1c_ref, d2a_ref, d2b_ref, d2c_ref,
                        key_ref, *, nsteps, NT, segs, Vcat):
    b = pl.program_id(0)
    ni = pl.program_id(1)
    g = gt_ref[0]
    p = p_ref[0]
    ab = jnp.dot(g, p, preferred_element_type=jnp.float32)
    aa = jnp.sum(g * g, axis=1, keepdims=True)
    bb = jnp.sum(p * p, axis=0, keepdims=True)
    D = jnp.maximum(aa + bb - 2.0 * ab, 0.0)
    d1refs = (d1a_ref, d1b_ref, d1c_ref)
    d2refs = (d2a_ref, d2b_ref, d2c_ref)

    @pl.when((b == 0) & (ni == 0))
    def _():
        for r in d1refs + d2refs:
            r[:, :] = jnp.zeros((1, 1), jnp.float32)

    for l, (o, w, V) in enumerate(segs):
        d1t = jnp.min(D[:, o:o + w], axis=1, keepdims=True)
        d1refs[l][:, :] += jnp.sum(jnp.sqrt(d1t + 1e-12), keepdims=True)

    rowi = lax.broadcasted_iota(jnp.int32, (NT, 1), 0) + ni * NT
    key = (lax.bitcast_convert_type(D, jnp.int32) & jnp.int32(-2048)) | rowi
    tkey = jnp.min(key, axis=0, keepdims=True)

    @pl.when(ni == 0)
    def _():
        key_ref[0] = tkey

    @pl.when(ni > 0)
    def _():
        key_ref[0] = jnp.minimum(key_ref[0], tkey)

    @pl.when(ni == nsteps - 1)
    def _():
        viota = lax.broadcasted_iota(jnp.int32, (1, Vcat), 1)
        kf = lax.bitcast_convert_type(key_ref[0] & jnp.int32(-2048),
                                      jnp.float32)
        d2sq = jnp.sqrt(kf + 1e-12)
        for l, (o, w, V) in enumerate(segs):
            valid = (viota[:, o:o + w] - o) < V
            seg = jnp.where(valid, d2sq[:, o:o + w], 0.0)
            d2refs[l][:, :] += jnp.sum(seg, keepdims=True)


def _chamfer_all(gt, preds):
    B, N, _ = gt.shape
    NT = 512
    nsteps = N // NT
    segs = []
    cats = []
    off = 0
    for P in preds:
        V = P.shape[1]
        Vp = max(256, ((V + 127) // 128) * 128)
        Pp = jnp.pad(P, ((0, 0), (0, Vp - V), (0, 0)), constant_values=_BIG)
        cats.append(jnp.transpose(Pp, (0, 2, 1)))
        segs.append((off, Vp, V))
        off += Vp
    Vcat = off
    Pt = jnp.concatenate(cats, axis=2)

    kern = functools.partial(_chamfer_all_kernel, nsteps=nsteps, NT=NT,
                             segs=tuple(segs), Vcat=Vcat)
    scal = pl.BlockSpec((1, 1), lambda b, n: (0, 0))
    outs = pl.pallas_call(
        kern,
        grid=(B, nsteps),
        in_specs=[
            pl.BlockSpec((1, NT, 3), lambda b, n: (b, n, 0)),
            pl.BlockSpec((1, 3, Vcat), lambda b, n: (b, 0, 0)),
        ],
        out_specs=[scal] * 6 + [
            pl.BlockSpec((1, 1, Vcat), lambda b, n: (b, 0, 0)),
        ],
        out_shape=[jax.ShapeDtypeStruct((1, 1), jnp.float32)] * 6 + [
            jax.ShapeDtypeStruct((B, 1, Vcat), jnp.int32),
        ],
    )(gt, Pt)
    d1_sums = [outs[l][0, 0] for l in range(3)]
    d2_sums = [outs[3 + l][0, 0] for l in range(3)]
    key_cat = outs[6][:, 0, :]
    return d1_sums, d2_sums, key_cat


_LVL = [
    dict(V=156, E=462, NCH=2, WBASE=24, EC=240, VC=80, Vr=160, Er=480, O=0),
    dict(V=618, E=1848, NCH=2, WBASE=16, EC=928, VC=320, Vr=640, Er=1856,
         O=256),
    dict(V=2466, E=7392, NCH=4, WBASE=0, EC=1856, VC=624, Vr=2496, Er=7424,
         O=896),
]
_VR_MAX = 2496
_EC_MAX = 1856
_VC_MAX = 624
_NB = 4
_NPTS = 2048
_VCAT = 3456

_off = 0
_PO = []
for _c in _LVL:
    _PO.append(_off)
    _off += 3 * _NB * _c["Vr"]
_BO = []
for _c in _LVL:
    _BO.append(_off)
    _off += 3 * _NB * _c["Vr"]
_NO = _off
_F_TOT = _NO + 3 * _NB * _NPTS
_off = 0
_LNO = []
for _c in _LVL:
    _LNO.append(_off)
    _off += 8 * _c["Vr"]
_LCO = []
for _c in _LVL:
    _LCO.append(_off)
    _off += _c["Vr"]
_ETO = []
for _c in _LVL:
    _ETO.append(_off)
    _off += 2 * _c["Er"]
_I_TOT = _off


def _rsqrt16(x):
    xi = plsc.bitcast(x, jnp.int32)
    yi = jnp.int32(0x5F3759DF) - (xi >> 1)
    y = plsc.bitcast(yi, jnp.float32)
    for _ in range(4):
        y = y * (1.5 - 0.5 * x * y * y)
    return y


def _rega_branch(cfg, lvl, wid, fb, ib,
                 out, pvs, bvs, dvs, e0v, e1v, lnv, lcv, ov, sem):
    V, E, NCH, WBASE = cfg["V"], cfg["E"], cfg["NCH"], cfg["WBASE"]
    EC, VC, Vr, Er = cfg["EC"], cfg["VC"], cfg["Vr"], cfg["Er"]
    PO, BO = _PO[lvl], _BO[lvl]
    LN, LC, ET = _LNO[lvl], _LCO[lvl], _ETO[lvl]
    B = _NB
    t = wid - WBASE
    b = t // NCH
    chunk = t % NCH

    cps = []
    for c in range(3):
        cps.append(pltpu.async_copy(fb.at[pl.ds(PO + (c * B + b) * Vr, Vr)],
                                    pvs[c].at[pl.ds(0, Vr)], sem))
        cps.append(pltpu.async_copy(fb.at[pl.ds(BO + (c * B + b) * Vr, Vr)],
                                    bvs[c].at[pl.ds(0, Vr)], sem))
    cps.append(pltpu.async_copy(ib.at[pl.ds(ET + chunk * EC, EC)],
                                e0v.at[pl.ds(0, EC)], sem))
    cps.append(pltpu.async_copy(ib.at[pl.ds(ET + Er + chunk * EC, EC)],
                                e1v.at[pl.ds(0, EC)], sem))
    for k in range(8):
        cps.append(pltpu.async_copy(
            ib.at[pl.ds(LN + k * Vr + chunk * VC, VC)],
            lnv.at[pl.ds(k * _VC_MAX, VC)], sem))
    cps.append(pltpu.async_copy(ib.at[pl.ds(LC + chunk * VC, VC)],
                                lcv.at[pl.ds(0, VC)], sem))
    for cp in cps:
        cp.wait()

    iota16 = lax.broadcasted_iota(jnp.int32, (16,), 0)
    z16 = jnp.zeros((16,), jnp.float32)

    def dv_body(i, carry):
        s = pl.ds(i * 16, 16)
        for c in range(3):
            dvs[c][s] = bvs[c][s] - pvs[c][s]
        return carry

    lax.fori_loop(0, Vr // 16, dv_body, 0)

    def e_body(i, ae):
        s = pl.ds(i * 16, 16)
        i0 = e0v[s]
        i1 = e1v[s]
        m = (chunk * EC + i * 16 + iota16) < E
        dx = plsc.load_gather(pvs[0], [i0]) - plsc.load_gather(pvs[0], [i1])
        dy = plsc.load_gather(pvs[1], [i0]) - plsc.load_gather(pvs[1], [i1])
        dz = plsc.load_gather(pvs[2], [i0]) - plsc.load_gather(pvs[2], [i1])
        dd = dx * dx + dy * dy + dz * dz
        return ae + jnp.where(m, dd, 0.0)

    ae = lax.fori_loop(0, EC // 16, e_body, z16)

    def l_body(i, carry):
        al, am = carry
        gbase = chunk * VC + i * 16
        sg = pl.ds(gbase, 16)
        m = (gbase + iota16) < V
        sx = dvs[0][sg]
        sy = dvs[1][sg]
        sz = dvs[2][sg]
        cntf = lcv[pl.ds(i * 16, 16)].astype(jnp.float32)
        accx = z16
        accy = z16
        accz = z16
        for k in range(8):
            nk = lnv[pl.ds(k * _VC_MAX + i * 16, 16)]
            mk = nk >= 0
            sk = jnp.where(mk, nk, 0)
            accx = accx + jnp.where(mk, plsc.load_gather(dvs[0], [sk]), 0.0)
            accy = accy + jnp.where(mk, plsc.load_gather(dvs[1], [sk]), 0.0)
            accz = accz + jnp.where(mk, plsc.load_gather(dvs[2], [sk]), 0.0)
        lx = sx - accx / cntf
        ly = sy - accy / cntf
        lz = sz - accz / cntf
        al = al + jnp.where(m, lx * lx + ly * ly + lz * lz, 0.0)
        am = am + jnp.where(m, sx * sx + sy * sy + sz * sz, 0.0)
        return al, am

    al, am = lax.fori_loop(0, VC // 16, l_body, (z16, z16))

    vec = (jnp.where(iota16 == lvl, jnp.sum(ae), 0.0)
           + jnp.where(iota16 == 6 + lvl, jnp.sum(al), 0.0)
           + jnp.where(iota16 == 9 + lvl, jnp.sum(am), 0.0))
    ov[...] = vec
    pltpu.sync_copy(ov, out.at[pl.ds(wid * 16, 16)])


def _regb_branch(cfg, lvl, wid, fb, ib, kf,
                 out, pvs, nvs, i2v, e0v, e1v, ov, sem):
    V, E, NCH, WBASE = cfg["V"], cfg["E"], cfg["NCH"], cfg["WBASE"]
    EC, Vr, Er, O = cfg["EC"], cfg["Vr"], cfg["Er"], cfg["O"]
    PO = _PO[lvl]
    ET = _ETO[lvl]
    B, N = _NB, _NPTS
    t = wid - WBASE
    b = t // NCH
    chunk = t % NCH

    cps = []
    for c in range(3):
        cps.append(pltpu.async_copy(fb.at[pl.ds(PO + (c * B + b) * Vr, Vr)],
                                    pvs[c].at[pl.ds(0, Vr)], sem))
        cps.append(pltpu.async_copy(fb.at[pl.ds(_NO + (c * B + b) * N, N)],
                                    nvs[c], sem))
    cps.append(pltpu.async_copy(kf.at[pl.ds(b * _VCAT + O, Vr)],
                                i2v.at[pl.ds(0, Vr)], sem))
    cps.append(pltpu.async_copy(ib.at[pl.ds(ET + chunk * EC, EC)],
                                e0v.at[pl.ds(0, EC)], sem))
    cps.append(pltpu.async_copy(ib.at[pl.ds(ET + Er + chunk * EC, EC)],
                                e1v.at[pl.ds(0, EC)], sem))
    for cp in cps:
        cp.wait()

    iota16 = lax.broadcasted_iota(jnp.int32, (16,), 0)
    z16 = jnp.zeros((16,), jnp.float32)

    def e_body(i, an):
        s = pl.ds(i * 16, 16)
        i0 = e0v[s]
        i1 = e1v[s]
        m = (chunk * EC + i * 16 + iota16) < E
        dx = plsc.load_gather(pvs[0], [i0]) - plsc.load_gather(pvs[0], [i1])
        dy = plsc.load_gather(pvs[1], [i0]) - plsc.load_gather(pvs[1], [i1])
        dz = plsc.load_gather(pvs[2], [i0]) - plsc.load_gather(pvs[2], [i1])
        dd = dx * dx + dy * dy + dz * dz
        j = plsc.load_gather(i2v, [i0]) & 2047
        gx = plsc.load_gather(nvs[0], [j])
        gy = plsc.load_gather(nvs[1], [j])
        gz = plsc.load_gather(nvs[2], [j])
        dot = dx * gx + dy * gy + dz * gz
        nn2 = gx * gx + gy * gy + gz * gz
        cos = jnp.abs(dot) * _rsqrt16(jnp.maximum(dd * nn2, 1e-24))
        return an + jnp.where(m, cos, 0.0)

    an = lax.fori_loop(0, EC // 16, e_body, z16)

    vec = jnp.where(iota16 == 3 + lvl, jnp.sum(an), 0.0)
    ov[...] = vec
    pltpu.sync_copy(ov, out.at[pl.ds(wid * 16, 16)])


def _rega_body(fb, ib, out,
               pvx, pvy, pvz, bvx, bvy, bvz, dvx, dvy, dvz,
               e0v, e1v, lnv, lcv, ov, sem):
    wid = lax.axis_index("c") * 16 + lax.axis_index("s")
    pvs = (pvx, pvy, pvz)
    bvs = (bvx, bvy, bvz)
    dvs = (dvx, dvy, dvz)
    for lvl in range(3):
        cfg = _LVL[lvl]
        lo = cfg["WBASE"]
        hi = lo + 4 * cfg["NCH"]

        @pl.when((wid >= lo) & (wid < hi))
        def _(lvl=lvl, cfg=cfg):
            _rega_branch(cfg, lvl, wid, fb, ib, out,
                         pvs, bvs, dvs, e0v, e1v, lnv, lcv, ov, sem)


def _regb_body(fb, ib, kf, out,
               pvx, pvy, pvz, nvx, nvy, nvz, i2v, e0v, e1v, ov, sem):
    wid = lax.axis_index("c") * 16 + lax.axis_index("s")
    pvs = (pvx, pvy, pvz)
    nvs = (nvx, nvy, nvz)
    for lvl in range(3):
        cfg = _LVL[lvl]
        lo = cfg["WBASE"]
        hi = lo + 4 * cfg["NCH"]

        @pl.when((wid >= lo) & (wid < hi))
        def _(lvl=lvl, cfg=cfg):
            _regb_branch(cfg, lvl, wid, fb, ib, kf, out,
                         pvs, nvs, i2v, e0v, e1v, ov, sem)


def _reg_pack(preds, befores, gt_normals, laps, edges_l):
    fparts = []
    iparts = []
    for lvl in range(3):
        cfg = _LVL[lvl]
        V, Vr = cfg["V"], cfg["Vr"]
        P = preds[lvl]
        p3 = jnp.transpose(jnp.pad(P, ((0, 0), (0, Vr - V), (0, 0))), (2, 0, 1))
        fparts.append(p3.reshape(-1))
    for lvl in range(3):
        cfg = _LVL[lvl]
        V, Vr = cfg["V"], cfg["Vr"]
        bf = befores[lvl]
        b3 = jnp.transpose(jnp.pad(bf, ((0, 0), (0, Vr - V), (0, 0))), (2, 0, 1))
        fparts.append(b3.reshape(-1))
    fparts.append(jnp.transpose(gt_normals, (2, 0, 1)).reshape(-1))
    for lvl in range(3):
        cfg = _LVL[lvl]
        V, Vr = cfg["V"], cfg["Vr"]
        iparts.append(jnp.pad(laps[lvl][:, :8].T, ((0, 0), (0, Vr - V))).reshape(-1))
    for lvl in range(3):
        cfg = _LVL[lvl]
        V, Vr = cfg["V"], cfg["Vr"]
        iparts.append(jnp.pad(laps[lvl][:, -1], (0, Vr - V), constant_values=1))
    for lvl in range(3):
        cfg = _LVL[lvl]
        E, Er = cfg["E"], cfg["Er"]
        iparts.append(jnp.pad(edges_l[lvl].T, ((0, 0), (0, Er - E))).reshape(-1))
    return jnp.concatenate(fparts), jnp.concatenate(iparts)


def _sc_mesh():
    return plsc.VectorSubcoreMesh(core_axis_name="c", subcore_axis_name="s",
                                  num_cores=2, num_subcores=16)


def _reg_sc_a(fb, ib):
    return pl.kernel(
        _rega_body,
        out_type=jax.ShapeDtypeStruct((512,), jnp.float32),
        mesh=_sc_mesh(),
        compiler_params=pltpu.CompilerParams(needs_layout_passes=False),
        scratch_types=(
            [pltpu.VMEM((_VR_MAX,), jnp.float32)] * 9
            + [pltpu.VMEM((_EC_MAX,), jnp.int32),
               pltpu.VMEM((_EC_MAX,), jnp.int32),
               pltpu.VMEM((8 * _VC_MAX,), jnp.int32),
               pltpu.VMEM((_VC_MAX,), jnp.int32),
               pltpu.VMEM((16,), jnp.float32),
               pltpu.SemaphoreType.DMA]
        ),
    )(fb, ib)


def _reg_sc_b(fb, ib, key_cat):
    return pl.kernel(
        _regb_body,
        out_type=jax.ShapeDtypeStruct((512,), jnp.float32),
        mesh=_sc_mesh(),
        compiler_params=pltpu.CompilerParams(needs_layout_passes=False),
        scratch_types=(
            [pltpu.VMEM((_VR_MAX,), jnp.float32)] * 3
            + [pltpu.VMEM((_NPTS,), jnp.float32)] * 3
            + [pltpu.VMEM((_VR_MAX,), jnp.int32),
               pltpu.VMEM((_EC_MAX,), jnp.int32),
               pltpu.VMEM((_EC_MAX,), jnp.int32),
               pltpu.VMEM((16,), jnp.float32),
               pltpu.SemaphoreType.DMA]
        ),
    )(fb, ib, key_cat.reshape(-1))


def kernel(pred_0, pred_1, pred_2, before_0, before_1, before_2,
           gt_points, gt_normals, gt_images,
           lap_idx_0, lap_idx_1, lap_idx_2,
           edges_0, edges_1, edges_2):
    w_chamfer_opp = 0.55
    w_laplace, w_move, w_edge, w_normal = 0.5, 0.1, 0.1, 0.00016
    lap_const = [0.2, 1.0, 1.0]
    preds = [pred_0, pred_1, pred_2]
    befores = [before_0, before_1, before_2]
    laps = [lap_idx_0, lap_idx_1, lap_idx_2]
    edges_l = [edges_0, edges_1, edges_2]
    B, N, _ = gt_points.shape

    fb, ib = _reg_pack(preds, befores, gt_normals, laps, edges_l)
    out_a = _reg_sc_a(fb, ib)

    chamfer_loss = jnp.float32(0.0)
    d1_sums, d2_sums, key_cat = _chamfer_all(gt_points, preds)
    for lvl in range(3):
        V = preds[lvl].shape[1]
        chamfer_loss = (chamfer_loss + d1_sums[lvl] / (B * N)
                        + w_chamfer_opp * d2_sums[lvl] / (B * V))

    out_b = _reg_sc_b(fb, ib, key_cat)
    sums = (out_a.reshape(32, 16) + out_b.reshape(32, 16)).sum(axis=0)

    edge_loss = jnp.float32(0.0)
    normal_loss = jnp.float32(0.0)
    lap_loss = jnp.float32(0.0)
    move_loss = jnp.float32(0.0)
    for lvl in range(3):
        V = preds[lvl].shape[1]
        E = edges_l[lvl].shape[0]
        edge_loss = edge_loss + sums[lvl] / (B * E)
        normal_loss = normal_loss + sums[3 + lvl] / (B * E)
        lap_loss = lap_loss + lap_const[lvl] * sums[6 + lvl] / (B * V)
        if lvl > 0:
            move_loss = move_loss + lap_const[lvl] * sums[9 + lvl] / (B * V)

    loss = (chamfer_loss + lap_loss * w_laplace + move_loss * w_move
            + edge_loss * w_edge + normal_loss * w_normal)
    return (loss, chamfer_loss, edge_loss, lap_loss, move_loss, normal_loss)


# speedup vs baseline: 1.0799x; 1.0684x over previous
import functools
import jax
import jax.numpy as jnp
from jax import lax
from jax.experimental import pallas as pl
from jax.experimental.pallas import tpu as pltpu
from jax.experimental.pallas import tpu_sc as plsc


_BIG = 1e6


def _chamfer_all_kernel(gt_ref, p_ref,
                        d1a_ref, d1b_ref, d1c_ref, d2a_ref, d2b_ref, d2c_ref,
                        key_ref, *, nsteps, NT, segs, Vcat):
    b = pl.program_id(0)
    ni = pl.program_id(1)
    g = gt_ref[0]
    p = p_ref[0]
    ab = jnp.dot(g, p, preferred_element_type=jnp.float32)
    aa = jnp.sum(g * g, axis=1, keepdims=True)
    bb = jnp.sum(p * p, axis=0, keepdims=True)
    D = jnp.maximum(aa + bb - 2.0 * ab, 0.0)
    d1refs = (d1a_ref, d1b_ref, d1c_ref)
    d2refs = (d2a_ref, d2b_ref, d2c_ref)

    @pl.when((b == 0) & (ni == 0))
    def _():
        for r in d1refs + d2refs:
            r[:, :] = jnp.zeros((1, 1), jnp.float32)

    for l, (o, w, V) in enumerate(segs):
        d1t = jnp.min(D[:, o:o + w], axis=1, keepdims=True)
        d1refs[l][:, :] += jnp.sum(jnp.sqrt(d1t + 1e-12), keepdims=True)

    rowi = lax.broadcasted_iota(jnp.int32, (NT, 1), 0) + ni * NT
    key = (lax.bitcast_convert_type(D, jnp.int32) & jnp.int32(-2048)) | rowi
    tkey = jnp.min(key, axis=0, keepdims=True)

    @pl.when(ni == 0)
    def _():
        key_ref[0] = tkey

    @pl.when(ni > 0)
    def _():
        key_ref[0] = jnp.minimum(key_ref[0], tkey)

    @pl.when(ni == nsteps - 1)
    def _():
        viota = lax.broadcasted_iota(jnp.int32, (1, Vcat), 1)
        kf = lax.bitcast_convert_type(key_ref[0] & jnp.int32(-2048),
                                      jnp.float32)
        d2sq = jnp.sqrt(kf + 1e-12)
        for l, (o, w, V) in enumerate(segs):
            valid = (viota[:, o:o + w] - o) < V
            seg = jnp.where(valid, d2sq[:, o:o + w], 0.0)
            d2refs[l][:, :] += jnp.sum(seg, keepdims=True)


def _chamfer_all(gt, preds):
    B, N, _ = gt.shape
    NT = 512
    nsteps = N // NT
    segs = []
    cats = []
    off = 0
    for P in preds:
        V = P.shape[1]
        Vp = max(256, ((V + 127) // 128) * 128)
        Pp = jnp.pad(P, ((0, 0), (0, Vp - V), (0, 0)), constant_values=_BIG)
        cats.append(jnp.transpose(Pp, (0, 2, 1)))
        segs.append((off, Vp, V))
        off += Vp
    Vcat = off
    Pt = jnp.concatenate(cats, axis=2)

    kern = functools.partial(_chamfer_all_kernel, nsteps=nsteps, NT=NT,
                             segs=tuple(segs), Vcat=Vcat)
    scal = pl.BlockSpec((1, 1), lambda b, n: (0, 0))
    outs = pl.pallas_call(
        kern,
        grid=(B, nsteps),
        in_specs=[
            pl.BlockSpec((1, NT, 3), lambda b, n: (b, n, 0)),
            pl.BlockSpec((1, 3, Vcat), lambda b, n: (b, 0, 0)),
        ],
        out_specs=[scal] * 6 + [
            pl.BlockSpec((1, 1, Vcat), lambda b, n: (b, 0, 0)),
        ],
        out_shape=[jax.ShapeDtypeStruct((1, 1), jnp.float32)] * 6 + [
            jax.ShapeDtypeStruct((B, 1, Vcat), jnp.int32),
        ],
    )(gt, Pt)
    d1_sums = [outs[l][0, 0] for l in range(3)]
    d2_sums = [outs[3 + l][0, 0] for l in range(3)]
    key_cat = outs[6][:, 0, :]
    return d1_sums, d2_sums, key_cat


_LVL = [
    dict(V=156, E=462, NCH=2, WBASE=24, EC=240, VC=80, Vr=160, Er=480, O=0),
    dict(V=618, E=1848, NCH=2, WBASE=16, EC=928, VC=320, Vr=640, Er=1856,
         O=256),
    dict(V=2466, E=7392, NCH=4, WBASE=0, EC=1856, VC=624, Vr=2496, Er=7424,
         O=896),
]
_VR_MAX = 2496
_EC_MAX = 1856
_VC_MAX = 624
_NB = 4
_NPTS = 2048
_VCAT = 3456

_off = 0
_PO = []
for _c in _LVL:
    _PO.append(_off)
    _off += 3 * _NB * _c["Vr"]
_BO = []
for _c in _LVL:
    _BO.append(_off)
    _off += 3 * _NB * _c["Vr"]
_NO = _off
_F_TOT = _NO + 3 * _NB * _NPTS
_off = 0
_LNO = []
for _c in _LVL:
    _LNO.append(_off)
    _off += 8 * _c["Vr"]
_LCO = []
for _c in _LVL:
    _LCO.append(_off)
    _off += _c["Vr"]
_ETO = []
for _c in _LVL:
    _ETO.append(_off)
    _off += 2 * _c["Er"]
_I_TOT = _off


def _rsqrt16(x):
    xi = plsc.bitcast(x, jnp.int32)
    yi = jnp.int32(0x5F3759DF) - (xi >> 1)
    y = plsc.bitcast(yi, jnp.float32)
    for _ in range(4):
        y = y * (1.5 - 0.5 * x * y * y)
    return y


def _reg_branch(cfg, lvl, wid, fb, ib, kf,
                out, pvs, bvs, dvs, nvs, i2v, e0v, e1v, lnv, lcv, ov, sem):
    V, E, NCH, WBASE = cfg["V"], cfg["E"], cfg["NCH"], cfg["WBASE"]
    EC, VC, Vr, Er, O = cfg["EC"], cfg["VC"], cfg["Vr"], cfg["Er"], cfg["O"]
    PO, BO = _PO[lvl], _BO[lvl]
    LN, LC, ET = _LNO[lvl], _LCO[lvl], _ETO[lvl]
    B, N = _NB, _NPTS
    t = wid - WBASE
    b = t // NCH
    chunk = t % NCH

    cps = []
    for c in range(3):
        cps.append(pltpu.async_copy(fb.at[pl.ds(PO + (c * B + b) * Vr, Vr)],
                                    pvs[c].at[pl.ds(0, Vr)], sem))
        cps.append(pltpu.async_copy(fb.at[pl.ds(BO + (c * B + b) * Vr, Vr)],
                                    bvs[c].at[pl.ds(0, Vr)], sem))
        cps.append(pltpu.async_copy(fb.at[pl.ds(_NO + (c * B + b) * N, N)],
                                    nvs[c], sem))
    cps.append(pltpu.async_copy(kf.at[pl.ds(b * _VCAT + O, Vr)],
                                i2v.at[pl.ds(0, Vr)], sem))
    cps.append(pltpu.async_copy(ib.at[pl.ds(ET + chunk * EC, EC)],
                                e0v.at[pl.ds(0, EC)], sem))
    cps.append(pltpu.async_copy(ib.at[pl.ds(ET + Er + chunk * EC, EC)],
                                e1v.at[pl.ds(0, EC)], sem))
    for k in range(8):
        cps.append(pltpu.async_copy(
            ib.at[pl.ds(LN + k * Vr + chunk * VC, VC)],
            lnv.at[pl.ds(k * _VC_MAX, VC)], sem))
    cps.append(pltpu.async_copy(ib.at[pl.ds(LC + chunk * VC, VC)],
                                lcv.at[pl.ds(0, VC)], sem))
    for cp in cps:
        cp.wait()

    iota16 = lax.broadcasted_iota(jnp.int32, (16,), 0)
    z16 = jnp.zeros((16,), jnp.float32)

    def dv_body(i, carry):
        s = pl.ds(i * 16, 16)
        for c in range(3):
            dvs[c][s] = bvs[c][s] - pvs[c][s]
        return carry

    lax.fori_loop(0, Vr // 16, dv_body, 0)

    def e_body(i, carry):
        ae, an = carry
        s = pl.ds(i * 16, 16)
        i0 = e0v[s]
        i1 = e1v[s]
        m = (chunk * EC + i * 16 + iota16) < E
        dx = plsc.load_gather(pvs[0], [i0]) - plsc.load_gather(pvs[0], [i1])
        dy = plsc.load_gather(pvs[1], [i0]) - plsc.load_gather(pvs[1], [i1])
        dz = plsc.load_gather(pvs[2], [i0]) - plsc.load_gather(pvs[2], [i1])
        dd = dx * dx + dy * dy + dz * dz
        j = plsc.load_gather(i2v, [i0]) & 2047
        gx = plsc.load_gather(nvs[0], [j])
        gy = plsc.load_gather(nvs[1], [j])
        gz = plsc.load_gather(nvs[2], [j])
        dot = dx * gx + dy * gy + dz * gz
        nn2 = gx * gx + gy * gy + gz * gz
        cos = jnp.abs(dot) * _rsqrt16(jnp.maximum(dd * nn2, 1e-24))
        ae = ae + jnp.where(m, dd, 0.0)
        an = an + jnp.where(m, cos, 0.0)
        return ae, an

    ae, an = lax.fori_loop(0, EC // 16, e_body, (z16, z16))

    def l_body(i, carry):
        al, am = carry
        gbase = chunk * VC + i * 16
        sg = pl.ds(gbase, 16)
        m = (gbase + iota16) < V
        sx = dvs[0][sg]
        sy = dvs[1][sg]
        sz = dvs[2][sg]
        cntf = lcv[pl.ds(i * 16, 16)].astype(jnp.float32)
        accx = z16
        accy = z16
        accz = z16
        for k in range(8):
            nk = lnv[pl.ds(k * _VC_MAX + i * 16, 16)]
            mk = nk >= 0
            sk = jnp.where(mk, nk, 0)
            accx = accx + jnp.where(mk, plsc.load_gather(dvs[0], [sk]), 0.0)
            accy = accy + jnp.where(mk, plsc.load_gather(dvs[1], [sk]), 0.0)
            accz = accz + jnp.where(mk, plsc.load_gather(dvs[2], [sk]), 0.0)
        lx = sx - accx / cntf
        ly = sy - accy / cntf
        lz = sz - accz / cntf
        al = al + jnp.where(m, lx * lx + ly * ly + lz * lz, 0.0)
        am = am + jnp.where(m, sx * sx + sy * sy + sz * sz, 0.0)
        return al, am

    al, am = lax.fori_loop(0, VC // 16, l_body, (z16, z16))

    vec = (jnp.where(iota16 == lvl, jnp.sum(ae), 0.0)
           + jnp.where(iota16 == 3 + lvl, jnp.sum(an), 0.0)
           + jnp.where(iota16 == 6 + lvl, jnp.sum(al), 0.0)
           + jnp.where(iota16 == 9 + lvl, jnp.sum(am), 0.0))
    ov[...] = vec
    pltpu.sync_copy(ov, out.at[pl.ds(wid * 16, 16)])


def _reg_body(fb, ib, kf, out,
              pvx, pvy, pvz, bvx, bvy, bvz, dvx, dvy, dvz,
              nvx, nvy, nvz, i2v, e0v, e1v, lnv, lcv, ov, sem):
    wid = lax.axis_index("c") * 16 + lax.axis_index("s")
    pvs = (pvx, pvy, pvz)
    bvs = (bvx, bvy, bvz)
    dvs = (dvx, dvy, dvz)
    nvs = (nvx, nvy, nvz)
    for lvl in range(3):
        cfg = _LVL[lvl]
        lo = cfg["WBASE"]
        hi = lo + 4 * cfg["NCH"]

        @pl.when((wid >= lo) & (wid < hi))
        def _(lvl=lvl, cfg=cfg):
            _reg_branch(cfg, lvl, wid, fb, ib, kf, out,
                        pvs, bvs, dvs, nvs, i2v, e0v, e1v, lnv, lcv, ov, sem)


def _reg_sc(preds, befores, gt_normals, key_cat, laps, edges_l):
    fparts = []
    iparts = []
    for lvl in range(3):
        cfg = _LVL[lvl]
        V, Vr = cfg["V"], cfg["Vr"]
        P = preds[lvl]
        p3 = jnp.transpose(jnp.pad(P, ((0, 0), (0, Vr - V), (0, 0))), (2, 0, 1))
        fparts.append(p3.reshape(-1))
    for lvl in range(3):
        cfg = _LVL[lvl]
        V, Vr = cfg["V"], cfg["Vr"]
        bf = befores[lvl]
        b3 = jnp.transpose(jnp.pad(bf, ((0, 0), (0, Vr - V), (0, 0))), (2, 0, 1))
        fparts.append(b3.reshape(-1))
    fparts.append(jnp.transpose(gt_normals, (2, 0, 1)).reshape(-1))
    for lvl in range(3):
        cfg = _LVL[lvl]
        V, Vr = cfg["V"], cfg["Vr"]
        iparts.append(jnp.pad(laps[lvl][:, :8].T, ((0, 0), (0, Vr - V))).reshape(-1))
    for lvl in range(3):
        cfg = _LVL[lvl]
        V, Vr = cfg["V"], cfg["Vr"]
        iparts.append(jnp.pad(laps[lvl][:, -1], (0, Vr - V), constant_values=1))
    for lvl in range(3):
        cfg = _LVL[lvl]
        E, Er = cfg["E"], cfg["Er"]
        iparts.append(jnp.pad(edges_l[lvl].T, ((0, 0), (0, Er - E))).reshape(-1))
    args = [jnp.concatenate(fparts), jnp.concatenate(iparts),
            key_cat.reshape(-1)]

    mesh = plsc.VectorSubcoreMesh(core_axis_name="c", subcore_axis_name="s",
                                  num_cores=2, num_subcores=16)
    out = pl.kernel(
        _reg_body,
        out_type=jax.ShapeDtypeStruct((512,), jnp.float32),
        mesh=mesh,
        compiler_params=pltpu.CompilerParams(needs_layout_passes=False),
        scratch_types=(
            [pltpu.VMEM((_VR_MAX,), jnp.float32)] * 9
            + [pltpu.VMEM((_NPTS,), jnp.float32)] * 3
            + [pltpu.VMEM((_VR_MAX,), jnp.int32),
               pltpu.VMEM((_EC_MAX,), jnp.int32),
               pltpu.VMEM((_EC_MAX,), jnp.int32),
               pltpu.VMEM((8 * _VC_MAX,), jnp.int32),
               pltpu.VMEM((_VC_MAX,), jnp.int32),
               pltpu.VMEM((16,), jnp.float32),
               pltpu.SemaphoreType.DMA]
        ),
    )(*args)
    sums = out.reshape(32, 16).sum(axis=0)
    return sums


def kernel(pred_0, pred_1, pred_2, before_0, before_1, before_2,
           gt_points, gt_normals, gt_images,
           lap_idx_0, lap_idx_1, lap_idx_2,
           edges_0, edges_1, edges_2):
    w_chamfer_opp = 0.55
    w_laplace, w_move, w_edge, w_normal = 0.5, 0.1, 0.1, 0.00016
    lap_const = [0.2, 1.0, 1.0]
    preds = [pred_0, pred_1, pred_2]
    befores = [before_0, before_1, before_2]
    laps = [lap_idx_0, lap_idx_1, lap_idx_2]
    edges_l = [edges_0, edges_1, edges_2]
    B, N, _ = gt_points.shape

    chamfer_loss = jnp.float32(0.0)
    d1_sums, d2_sums, key_cat = _chamfer_all(gt_points, preds)
    for lvl in range(3):
        V = preds[lvl].shape[1]
        chamfer_loss = (chamfer_loss + d1_sums[lvl] / (B * N)
                        + w_chamfer_opp * d2_sums[lvl] / (B * V))

    sums = _reg_sc(preds, befores, gt_normals, key_cat, laps, edges_l)

    edge_loss = jnp.float32(0.0)
    normal_loss = jnp.float32(0.0)
    lap_loss = jnp.float32(0.0)
    move_loss = jnp.float32(0.0)
    for lvl in range(3):
        V = preds[lvl].shape[1]
        E = edges_l[lvl].shape[0]
        edge_loss = edge_loss + sums[lvl] / (B * E)
        normal_loss = normal_loss + sums[3 + lvl] / (B * E)
        lap_loss = lap_loss + lap_const[lvl] * sums[6 + lvl] / (B * V)
        if lvl > 0:
            move_loss = move_loss + lap_const[lvl] * sums[9 + lvl] / (B * V)

    loss = (chamfer_loss + lap_loss * w_laplace + move_loss * w_move
            + edge_loss * w_edge + normal_loss * w_normal)
    return (loss, chamfer_loss, edge_loss, lap_loss, move_loss, normal_loss)
